# Initial kernel scaffold; baseline (speedup 1.0000x reference)
import jax
import jax.numpy as jnp
from jax.experimental import pallas as pl


def kernel(x, embedding):
    raise NotImplementedError("write your pallas kernel here")


# speedup vs baseline: 2.7589x; 2.7589x over previous
import jax
import jax.numpy as jnp
from jax.experimental import pallas as pl


def _copy_block(in_ref, out_ref):
    out_ref[...] = in_ref[...]


def kernel(x, embedding):
    seq_len = x.shape[1]
    features = embedding.shape[1]
    block = 512
    return pl.pallas_call(
        _copy_block,
        grid=(seq_len // block,),
        in_specs=[pl.BlockSpec((block, features), lambda i: (i, 0))],
        out_specs=pl.BlockSpec((block, features), lambda i: (i, 0)),
        out_shape=jax.ShapeDtypeStruct((seq_len, features), embedding.dtype),
    )(embedding)
